# baseline (speedup 1.0000x reference)
import functools

import jax
import jax.numpy as jnp
from jax import lax
from jax.experimental import pallas as pl
from jax.experimental.pallas import tpu as pltpu
from jax.experimental.pallas import tpu_sc as plsc

NC = 2
NS = 16


def _chunk(n, cap=128):
    for k in range(cap - cap % 8, 0, -8):
        if n % k == 0:
            return k
    raise ValueError(f"no multiple-of-8 chunk for {n}")


def _div(n, cap=128):
    for k in range(min(cap, n), 0, -1):
        if n % k == 0:
            return k
    return 1


def _sc_degree(ei_flat, zeros1, ones1, npad):
    e = ei_flat.shape[0] // 2
    e_per = e // (NC * NS)
    k = 40
    nbuf = 5
    nloops = e_per // k
    ngrp = nloops // nbuf
    stripe = npad // NS
    mesh = plsc.VectorSubcoreMesh(core_axis_name="c", subcore_axis_name="s")

    @functools.partial(
        pl.kernel,
        out_type=jax.ShapeDtypeStruct((NC, npad), jnp.float32),
        mesh=mesh,
        scratch_types=[
            pltpu.VMEM_SHARED((npad,), jnp.float32),
            pltpu.VMEM((k,), jnp.float32),
            pltpu.VMEM((nbuf, k), jnp.int32),
            [pltpu.SemaphoreType.DMA] * nbuf,
            [pltpu.SemaphoreType.DMA] * nbuf,
        ],
    )
    def deg_kernel(ei_hbm, zeros_hbm, ones_hbm, out_hbm,
                   accum, ones_v, didx, dis, sss):
        c = lax.axis_index("c")
        s = lax.axis_index("s")
        pltpu.sync_copy(zeros_hbm.at[pl.ds(s * stripe, stripe)],
                        accum.at[pl.ds(s * stripe, stripe)])
        pltpu.sync_copy(ones_hbm, ones_v)
        base = e + (c * NS + s) * e_per

        def idx_copy(j, b):
            pltpu.async_copy(ei_hbm.at[pl.ds(base + j * k, k)], didx.at[b], dis[b])

        def wait_idx(b):
            pltpu.make_async_copy(ei_hbm.at[pl.ds(base, k)], didx.at[b], dis[b]).wait()

        def scat(b):
            pltpu.async_copy(ones_v, accum.at[didx.at[b]], sss[b], add=True)

        def wait_scat(b):
            pltpu.make_async_copy(ones_v, accum.at[didx.at[b]], sss[b]).wait()

        plsc.subcore_barrier()
        for b in range(nbuf):
            idx_copy(b, b)

        def outer(g, carry):
            for b in range(nbuf):
                wait_idx(b)
                scat(b)
            for b in range(nbuf):
                wait_scat(b)
                idx_copy((g + 1) * nbuf + b, b)
            return carry

        lax.fori_loop(0, ngrp - 1, outer, 0)
        for b in range(nbuf):
            wait_idx(b)
            scat(b)
        for b in range(nbuf):
            wait_scat(b)
        plsc.subcore_barrier()
        pltpu.sync_copy(accum.at[pl.ds(s * stripe, stripe)],
                        out_hbm.at[c].at[pl.ds(s * stripe, stripe)])

    return deg_kernel(ei_flat, zeros1, ones1)


def _sc_scatter_add(vh, ei_flat, zeros_h, npad):
    e = ei_flat.shape[0] // 2
    h = vh.shape[2]
    e_per = e // NS
    k = 80
    nloops = e_per // k
    ngrp = (nloops - 3) // 2
    stripe = npad // NS
    mesh = plsc.VectorSubcoreMesh(core_axis_name="c", subcore_axis_name="s")

    @functools.partial(
        pl.kernel,
        out_type=jax.ShapeDtypeStruct((NC, npad, h), jnp.float32),
        mesh=mesh,
        scratch_types=[
            pltpu.VMEM_SHARED((npad, h), jnp.float32),
            pltpu.VMEM((2, k, h), jnp.float32),
            pltpu.VMEM((2, k), jnp.int32),
            pltpu.VMEM((2, k), jnp.int32),
            [pltpu.SemaphoreType.DMA] * 2,
            [pltpu.SemaphoreType.DMA] * 2,
            [pltpu.SemaphoreType.DMA] * 2,
            [pltpu.SemaphoreType.DMA] * 2,
        ],
    )
    def scat_kernel(vh_hbm, ei_hbm, zeros_hbm, out_hbm,
                    accum, rows, sidx, didx, sis, dis, gss, sss):
        c = lax.axis_index("c")
        s = lax.axis_index("s")
        pltpu.sync_copy(zeros_hbm.at[pl.ds(s * stripe, stripe)],
                        accum.at[pl.ds(s * stripe, stripe)])
        base = s * e_per

        def idx_copy(j, b):
            off = base + j * k
            pltpu.async_copy(ei_hbm.at[pl.ds(off, k)], sidx.at[b], sis[b])
            pltpu.async_copy(ei_hbm.at[pl.ds(e + off, k)], didx.at[b], dis[b])

        def wait_sidx(b):
            pltpu.make_async_copy(ei_hbm.at[pl.ds(base, k)], sidx.at[b], sis[b]).wait()

        def wait_didx(b):
            pltpu.make_async_copy(ei_hbm.at[pl.ds(base, k)], didx.at[b], dis[b]).wait()

        def gather(b):
            pltpu.async_copy(vh_hbm.at[c].at[sidx.at[b]], rows.at[b], gss[b])

        def wait_gather(b):
            pltpu.make_async_copy(vh_hbm.at[c].at[sidx.at[b]], rows.at[b], gss[b]).wait()

        def scat(b):
            pltpu.async_copy(rows.at[b], accum.at[didx.at[b]], sss[b], add=True)

        def wait_scat(b):
            pltpu.make_async_copy(rows.at[b], accum.at[didx.at[b]], sss[b]).wait()

        plsc.subcore_barrier()
        for b in range(2):
            idx_copy(b, b)
        for b in range(2):
            wait_sidx(b)
            gather(b)

        def outer(g, carry):
            for b in range(2):
                wait_gather(b)
                wait_didx(b)
                scat(b)
            for b in range(2):
                wait_scat(b)
                idx_copy(2 * g + 2 + b, b)
            for b in range(2):
                wait_sidx(b)
                gather(b)
            return carry

        lax.fori_loop(0, ngrp, outer, 0)
        last = nloops - 1
        wait_gather(0)
        wait_didx(0)
        scat(0)
        wait_scat(0)
        idx_copy(last, 0)
        wait_gather(1)
        wait_didx(1)
        scat(1)
        wait_sidx(0)
        gather(0)
        wait_gather(0)
        wait_didx(0)
        scat(0)
        for b in range(2):
            wait_scat(b)
        plsc.subcore_barrier()
        pltpu.sync_copy(accum.at[pl.ds(s * stripe, stripe)],
                        out_hbm.at[c].at[pl.ds(s * stripe, stripe)])

    return scat_kernel(vh, ei_flat, zeros_h)


def _dinv_from(d_ref):
    deg = d_ref[0] + d_ref[1] + 1.0
    return lax.rsqrt(jnp.maximum(deg, 1e-12))


def _tc1_body(x_ref, w_ref, d_ref, o_ref):
    dinv = _dinv_from(d_ref)
    u = jnp.dot(x_ref[...], w_ref[...], preferred_element_type=jnp.float32)
    o_ref[...] = (dinv * u)[None]


def _tc1(x, w1, deg2):
    n, f = x.shape
    h = f // 2
    r = 1000
    return pl.pallas_call(
        _tc1_body,
        grid=(NC, n // r),
        in_specs=[
            pl.BlockSpec((r, f), lambda c, i: (i, 0)),
            pl.BlockSpec((f, h), lambda c, i: (0, c)),
            pl.BlockSpec((NC, r, 1), lambda c, i: (0, i, 0)),
        ],
        out_specs=pl.BlockSpec((1, r, h), lambda c, i: (c, i, 0)),
        out_shape=jax.ShapeDtypeStruct((NC, n, h), jnp.float32),
    )(x, w1, deg2)


def _tc2_body(s1_ref, v_ref, d_ref, b_ref, o_ref):
    dinv = _dinv_from(d_ref)
    hact = jnp.maximum(dinv * (s1_ref[0] + v_ref[0]) + b_ref[0], 0.0)
    o_ref[...] = (dinv * hact)[None]


def _tc2(s1, v, deg2, b1r):
    _, n, h = v.shape
    r = 1000
    return pl.pallas_call(
        _tc2_body,
        grid=(NC, n // r),
        in_specs=[
            pl.BlockSpec((1, r, h), lambda c, i: (c, i, 0)),
            pl.BlockSpec((1, r, h), lambda c, i: (c, i, 0)),
            pl.BlockSpec((NC, r, 1), lambda c, i: (0, i, 0)),
            pl.BlockSpec((1, 1, h), lambda c, i: (c, 0, 0)),
        ],
        out_specs=pl.BlockSpec((1, r, h), lambda c, i: (c, i, 0)),
        out_shape=jax.ShapeDtypeStruct((NC, n, h), jnp.float32),
    )(s1, v, deg2, b1r)


def _tc3_body(s2_ref, v2_ref, d_ref, w2_ref, w3_ref, b2_ref, b3_ref,
              mu_ref, ls_ref):
    dinv = _dinv_from(d_ref)
    g0 = dinv * (s2_ref[0] + v2_ref[0])
    g1 = dinv * (s2_ref[1] + v2_ref[1])
    mu_ref[...] = (jnp.dot(g0, w2_ref[0], preferred_element_type=jnp.float32)
                   + jnp.dot(g1, w2_ref[1], preferred_element_type=jnp.float32)
                   + b2_ref[...])
    ls_ref[...] = (jnp.dot(g0, w3_ref[0], preferred_element_type=jnp.float32)
                   + jnp.dot(g1, w3_ref[1], preferred_element_type=jnp.float32)
                   + b3_ref[...])


def _tc3(s2, v2, deg2, w2r, w3r, b2r, b3r):
    _, n, h = v2.shape
    r = 1000
    return pl.pallas_call(
        _tc3_body,
        grid=(n // r,),
        in_specs=[
            pl.BlockSpec((NC, r, h), lambda i: (0, i, 0)),
            pl.BlockSpec((NC, r, h), lambda i: (0, i, 0)),
            pl.BlockSpec((NC, r, 1), lambda i: (0, i, 0)),
            pl.BlockSpec((NC, h, h), lambda i: (0, 0, 0)),
            pl.BlockSpec((NC, h, h), lambda i: (0, 0, 0)),
            pl.BlockSpec((1, h), lambda i: (0, 0)),
            pl.BlockSpec((1, h), lambda i: (0, 0)),
        ],
        out_specs=[
            pl.BlockSpec((r, h), lambda i: (i, 0)),
            pl.BlockSpec((r, h), lambda i: (i, 0)),
        ],
        out_shape=[
            jax.ShapeDtypeStruct((n, h), jnp.float32),
            jax.ShapeDtypeStruct((n, h), jnp.float32),
        ],
    )(s2, v2, deg2, w2r, w3r, b2r, b3r)


def kernel(x, edge_idx, W1, b1, W2, b2, W3, b3):
    n, f = x.shape
    h = f // 2
    ei_flat = edge_idx.astype(jnp.int32).reshape(-1)
    e = ei_flat.shape[0] // 2

    npad = -(-n // (NS * 128)) * (NS * 128)
    zeros_h = jnp.zeros((npad, h), jnp.float32)
    zeros1 = jnp.zeros((npad,), jnp.float32)
    ones1 = jnp.ones((40,), jnp.float32)
    b1r = b1.reshape(NC, 1, h)
    w2r = W2.reshape(NC, h, h)
    w3r = W3.reshape(NC, h, h)
    b2r = b2.reshape(1, h)
    b3r = b3.reshape(1, h)

    deg2 = _sc_degree(ei_flat, zeros1, ones1, npad).reshape(NC, npad, 1)
    v = _tc1(x, W1, deg2)
    s1 = _sc_scatter_add(v, ei_flat, zeros_h, npad)
    v2 = _tc2(s1, v, deg2, b1r)
    s2 = _sc_scatter_add(v2, ei_flat, zeros_h, npad)
    mu, log_std = _tc3(s2, v2, deg2, w2r, w3r, b2r, b3r)
    return (mu, log_std)


# speedup vs baseline: 1.5601x; 1.5601x over previous
import functools

import jax
import jax.numpy as jnp
from jax import lax
from jax.experimental import pallas as pl
from jax.experimental.pallas import tpu as pltpu
from jax.experimental.pallas import tpu_sc as plsc

NC = 2
NS = 16


def _chunk(n, cap=128):
    for k in range(cap - cap % 8, 0, -8):
        if n % k == 0:
            return k
    raise ValueError(f"no multiple-of-8 chunk for {n}")


def _div(n, cap=128):
    for k in range(min(cap, n), 0, -1):
        if n % k == 0:
            return k
    return 1


def _sc_degree(ei_flat, zeros1, ones1, npad):
    e = ei_flat.shape[0] // 2
    e_per = e // (NC * NS)
    k = 40
    nbuf = 5
    nloops = e_per // k
    ngrp = nloops // nbuf
    stripe = npad // NS
    mesh = plsc.VectorSubcoreMesh(core_axis_name="c", subcore_axis_name="s")

    @functools.partial(
        pl.kernel,
        out_type=jax.ShapeDtypeStruct((NC, npad), jnp.float32),
        mesh=mesh,
        scratch_types=[
            pltpu.VMEM_SHARED((npad,), jnp.float32),
            pltpu.VMEM((k,), jnp.float32),
            pltpu.VMEM((nbuf, k), jnp.int32),
            [pltpu.SemaphoreType.DMA] * nbuf,
            [pltpu.SemaphoreType.DMA] * nbuf,
        ],
    )
    def deg_kernel(ei_hbm, zeros_hbm, ones_hbm, out_hbm,
                   accum, ones_v, didx, dis, sss):
        c = lax.axis_index("c")
        s = lax.axis_index("s")
        pltpu.sync_copy(zeros_hbm.at[pl.ds(s * stripe, stripe)],
                        accum.at[pl.ds(s * stripe, stripe)])
        pltpu.sync_copy(ones_hbm, ones_v)
        base = e + (c * NS + s) * e_per

        def idx_copy(j, b):
            pltpu.async_copy(ei_hbm.at[pl.ds(base + j * k, k)], didx.at[b], dis[b])

        def wait_idx(b):
            pltpu.make_async_copy(ei_hbm.at[pl.ds(base, k)], didx.at[b], dis[b]).wait()

        def scat(b):
            pltpu.async_copy(ones_v, accum.at[didx.at[b]], sss[b], add=True)

        def wait_scat(b):
            pltpu.make_async_copy(ones_v, accum.at[didx.at[b]], sss[b]).wait()

        plsc.subcore_barrier()
        for b in range(nbuf):
            idx_copy(b, b)

        def outer(g, carry):
            for b in range(nbuf):
                wait_idx(b)
                scat(b)
            for b in range(nbuf):
                wait_scat(b)
                idx_copy((g + 1) * nbuf + b, b)
            return carry

        lax.fori_loop(0, ngrp - 1, outer, 0)
        for b in range(nbuf):
            wait_idx(b)
            scat(b)
        for b in range(nbuf):
            wait_scat(b)
        plsc.subcore_barrier()
        pltpu.sync_copy(accum.at[pl.ds(s * stripe, stripe)],
                        out_hbm.at[c].at[pl.ds(s * stripe, stripe)])

    return deg_kernel(ei_flat, zeros1, ones1)


def _sc_scatter_add(vh, ei_flat, zeros_h, npad):
    e = ei_flat.shape[0] // 2
    h = vh.shape[2]
    e_per = e // NS
    k = 40
    nr = 5
    ni = 10
    nloops = e_per // k
    ngrp = nloops // ni - 1
    stripe = npad // NS
    mesh = plsc.VectorSubcoreMesh(core_axis_name="c", subcore_axis_name="s")

    @functools.partial(
        pl.kernel,
        out_type=jax.ShapeDtypeStruct((NC, npad, h), jnp.float32),
        mesh=mesh,
        scratch_types=[
            pltpu.VMEM_SHARED((npad, h), jnp.float32),
            pltpu.VMEM((nr, k, h), jnp.float32),
            pltpu.VMEM((ni, k), jnp.int32),
            pltpu.VMEM((ni, k), jnp.int32),
            [pltpu.SemaphoreType.DMA] * ni,
            [pltpu.SemaphoreType.DMA] * ni,
            [pltpu.SemaphoreType.DMA] * nr,
            [pltpu.SemaphoreType.DMA] * nr,
        ],
    )
    def scat_kernel(vh_hbm, ei_hbm, zeros_hbm, out_hbm,
                    accum, rows, sidx, didx, sis, dis, gss, sss):
        c = lax.axis_index("c")
        s = lax.axis_index("s")
        pltpu.sync_copy(zeros_hbm.at[pl.ds(s * stripe, stripe)],
                        accum.at[pl.ds(s * stripe, stripe)])
        base = s * e_per

        def idx_copy(j, q):
            off = base + j * k
            pltpu.async_copy(ei_hbm.at[pl.ds(off, k)], sidx.at[q], sis[q])
            pltpu.async_copy(ei_hbm.at[pl.ds(e + off, k)], didx.at[q], dis[q])

        def wait_sidx(q):
            pltpu.make_async_copy(ei_hbm.at[pl.ds(base, k)], sidx.at[q], sis[q]).wait()

        def wait_didx(q):
            pltpu.make_async_copy(ei_hbm.at[pl.ds(base, k)], didx.at[q], dis[q]).wait()

        def gather(q, b):
            pltpu.async_copy(vh_hbm.at[c].at[sidx.at[q]], rows.at[b], gss[b])

        def wait_gather(q, b):
            pltpu.make_async_copy(vh_hbm.at[c].at[sidx.at[q]], rows.at[b], gss[b]).wait()

        def scat(q, b):
            pltpu.async_copy(rows.at[b], accum.at[didx.at[q]], sss[b], add=True)

        def wait_scat(q, b):
            pltpu.make_async_copy(rows.at[b], accum.at[didx.at[q]], sss[b]).wait()

        plsc.subcore_barrier()
        for q in range(ni):
            idx_copy(q, q)
        for b in range(nr):
            wait_sidx(b)
            gather(b, b)

        def outer(g, carry):
            j0 = g * ni
            for t in range(ni):
                q = t
                b = t % nr
                qn = (t + nr) % ni
                wait_gather(q, b)
                wait_didx(q)
                scat(q, b)
                wait_scat(q, b)
                idx_copy(j0 + t + ni, q)
                wait_sidx(qn)
                gather(qn, b)
            return carry

        lax.fori_loop(0, ngrp, outer, 0)
        for t in range(ni):
            q = t
            b = t % nr
            qn = (t + nr) % ni
            wait_gather(q, b)
            wait_didx(q)
            scat(q, b)
            wait_scat(q, b)
            if t < nr:
                wait_sidx(qn)
                gather(qn, b)
        plsc.subcore_barrier()
        pltpu.sync_copy(accum.at[pl.ds(s * stripe, stripe)],
                        out_hbm.at[c].at[pl.ds(s * stripe, stripe)])

    return scat_kernel(vh, ei_flat, zeros_h)


def _dinv_from(d_ref):
    deg = d_ref[0] + d_ref[1] + 1.0
    return lax.rsqrt(jnp.maximum(deg, 1e-12))


def _tc1_body(x_ref, w_ref, d_ref, o_ref):
    dinv = _dinv_from(d_ref)
    u = jnp.dot(x_ref[...], w_ref[...], preferred_element_type=jnp.float32)
    o_ref[...] = (dinv * u)[None]


def _tc1(x, w1, deg2):
    n, f = x.shape
    h = f // 2
    r = 1000
    return pl.pallas_call(
        _tc1_body,
        grid=(NC, n // r),
        in_specs=[
            pl.BlockSpec((r, f), lambda c, i: (i, 0)),
            pl.BlockSpec((f, h), lambda c, i: (0, c)),
            pl.BlockSpec((NC, r, 1), lambda c, i: (0, i, 0)),
        ],
        out_specs=pl.BlockSpec((1, r, h), lambda c, i: (c, i, 0)),
        out_shape=jax.ShapeDtypeStruct((NC, n, h), jnp.float32),
    )(x, w1, deg2)


def _tc2_body(s1_ref, v_ref, d_ref, b_ref, o_ref):
    dinv = _dinv_from(d_ref)
    hact = jnp.maximum(dinv * (s1_ref[0] + v_ref[0]) + b_ref[0], 0.0)
    o_ref[...] = (dinv * hact)[None]


def _tc2(s1, v, deg2, b1r):
    _, n, h = v.shape
    r = 1000
    return pl.pallas_call(
        _tc2_body,
        grid=(NC, n // r),
        in_specs=[
            pl.BlockSpec((1, r, h), lambda c, i: (c, i, 0)),
            pl.BlockSpec((1, r, h), lambda c, i: (c, i, 0)),
            pl.BlockSpec((NC, r, 1), lambda c, i: (0, i, 0)),
            pl.BlockSpec((1, 1, h), lambda c, i: (c, 0, 0)),
        ],
        out_specs=pl.BlockSpec((1, r, h), lambda c, i: (c, i, 0)),
        out_shape=jax.ShapeDtypeStruct((NC, n, h), jnp.float32),
    )(s1, v, deg2, b1r)


def _tc3_body(s2_ref, v2_ref, d_ref, w2_ref, w3_ref, b2_ref, b3_ref,
              mu_ref, ls_ref):
    dinv = _dinv_from(d_ref)
    g0 = dinv * (s2_ref[0] + v2_ref[0])
    g1 = dinv * (s2_ref[1] + v2_ref[1])
    mu_ref[...] = (jnp.dot(g0, w2_ref[0], preferred_element_type=jnp.float32)
                   + jnp.dot(g1, w2_ref[1], preferred_element_type=jnp.float32)
                   + b2_ref[...])
    ls_ref[...] = (jnp.dot(g0, w3_ref[0], preferred_element_type=jnp.float32)
                   + jnp.dot(g1, w3_ref[1], preferred_element_type=jnp.float32)
                   + b3_ref[...])


def _tc3(s2, v2, deg2, w2r, w3r, b2r, b3r):
    _, n, h = v2.shape
    r = 1000
    return pl.pallas_call(
        _tc3_body,
        grid=(n // r,),
        in_specs=[
            pl.BlockSpec((NC, r, h), lambda i: (0, i, 0)),
            pl.BlockSpec((NC, r, h), lambda i: (0, i, 0)),
            pl.BlockSpec((NC, r, 1), lambda i: (0, i, 0)),
            pl.BlockSpec((NC, h, h), lambda i: (0, 0, 0)),
            pl.BlockSpec((NC, h, h), lambda i: (0, 0, 0)),
            pl.BlockSpec((1, h), lambda i: (0, 0)),
            pl.BlockSpec((1, h), lambda i: (0, 0)),
        ],
        out_specs=[
            pl.BlockSpec((r, h), lambda i: (i, 0)),
            pl.BlockSpec((r, h), lambda i: (i, 0)),
        ],
        out_shape=[
            jax.ShapeDtypeStruct((n, h), jnp.float32),
            jax.ShapeDtypeStruct((n, h), jnp.float32),
        ],
    )(s2, v2, deg2, w2r, w3r, b2r, b3r)


def kernel(x, edge_idx, W1, b1, W2, b2, W3, b3):
    n, f = x.shape
    h = f // 2
    ei_flat = edge_idx.astype(jnp.int32).reshape(-1)
    e = ei_flat.shape[0] // 2

    npad = -(-n // (NS * 128)) * (NS * 128)
    zeros_h = jnp.zeros((npad, h), jnp.float32)
    zeros1 = jnp.zeros((npad,), jnp.float32)
    ones1 = jnp.ones((40,), jnp.float32)
    b1r = b1.reshape(NC, 1, h)
    w2r = W2.reshape(NC, h, h)
    w3r = W3.reshape(NC, h, h)
    b2r = b2.reshape(1, h)
    b3r = b3.reshape(1, h)

    deg2 = _sc_degree(ei_flat, zeros1, ones1, npad).reshape(NC, npad, 1)
    v = _tc1(x, W1, deg2)
    s1 = _sc_scatter_add(v, ei_flat, zeros_h, npad)
    v2 = _tc2(s1, v, deg2, b1r)
    s2 = _sc_scatter_add(v2, ei_flat, zeros_h, npad)
    mu, log_std = _tc3(s2, v2, deg2, w2r, w3r, b2r, b3r)
    return (mu, log_std)


# speedup vs baseline: 1.5720x; 1.0076x over previous
import functools

import jax
import jax.numpy as jnp
from jax import lax
from jax.experimental import pallas as pl
from jax.experimental.pallas import tpu as pltpu
from jax.experimental.pallas import tpu_sc as plsc

NC = 2
NS = 16


def _chunk(n, cap=128):
    for k in range(cap - cap % 8, 0, -8):
        if n % k == 0:
            return k
    raise ValueError(f"no multiple-of-8 chunk for {n}")


def _div(n, cap=128):
    for k in range(min(cap, n), 0, -1):
        if n % k == 0:
            return k
    return 1


def _sc_degree(ei_flat, zeros1, ones1, npad):
    e = ei_flat.shape[0] // 2
    e_per = e // (NC * NS)
    k = 40
    nbuf = 5
    nloops = e_per // k
    ngrp = nloops // nbuf
    stripe = npad // NS
    mesh = plsc.VectorSubcoreMesh(core_axis_name="c", subcore_axis_name="s")

    @functools.partial(
        pl.kernel,
        out_type=jax.ShapeDtypeStruct((NC, npad), jnp.float32),
        mesh=mesh,
        scratch_types=[
            pltpu.VMEM_SHARED((npad,), jnp.float32),
            pltpu.VMEM((k,), jnp.float32),
            pltpu.VMEM((nbuf, k), jnp.int32),
            [pltpu.SemaphoreType.DMA] * nbuf,
            [pltpu.SemaphoreType.DMA] * nbuf,
        ],
    )
    def deg_kernel(ei_hbm, zeros_hbm, ones_hbm, out_hbm,
                   accum, ones_v, didx, dis, sss):
        c = lax.axis_index("c")
        s = lax.axis_index("s")
        base = e + (c * NS + s) * e_per

        def idx_copy(j, b):
            pltpu.async_copy(ei_hbm.at[pl.ds(base + j * k, k)], didx.at[b], dis[b])

        def wait_idx(b):
            pltpu.make_async_copy(ei_hbm.at[pl.ds(base, k)], didx.at[b], dis[b]).wait()

        def scat(b):
            pltpu.async_copy(ones_v, accum.at[didx.at[b]], sss[b], add=True)

        def wait_scat(b):
            pltpu.make_async_copy(ones_v, accum.at[didx.at[b]], sss[b]).wait()

        for b in range(nbuf):
            idx_copy(b, b)
        pltpu.sync_copy(zeros_hbm.at[pl.ds(s * stripe, stripe)],
                        accum.at[pl.ds(s * stripe, stripe)])
        pltpu.sync_copy(ones_hbm, ones_v)
        plsc.subcore_barrier()

        def outer(g, carry):
            for b in range(nbuf):
                wait_idx(b)
                scat(b)
            for b in range(nbuf):
                wait_scat(b)
                idx_copy((g + 1) * nbuf + b, b)
            return carry

        lax.fori_loop(0, ngrp - 1, outer, 0)
        for b in range(nbuf):
            wait_idx(b)
            scat(b)
        for b in range(nbuf):
            wait_scat(b)
        plsc.subcore_barrier()
        pltpu.sync_copy(accum.at[pl.ds(s * stripe, stripe)],
                        out_hbm.at[c].at[pl.ds(s * stripe, stripe)])

    return deg_kernel(ei_flat, zeros1, ones1)


def _sc_scatter_add(vh, ei_flat, zeros_h, npad):
    e = ei_flat.shape[0] // 2
    h = vh.shape[2]
    e_per = e // NS
    k = 40
    nr = 5
    ni = 10
    nloops = e_per // k
    ngrp = nloops // ni - 1
    stripe = npad // NS
    mesh = plsc.VectorSubcoreMesh(core_axis_name="c", subcore_axis_name="s")

    @functools.partial(
        pl.kernel,
        out_type=jax.ShapeDtypeStruct((NC, npad, h), jnp.float32),
        mesh=mesh,
        scratch_types=[
            pltpu.VMEM_SHARED((npad, h), jnp.float32),
            pltpu.VMEM((nr, k, h), jnp.float32),
            pltpu.VMEM((ni, k), jnp.int32),
            pltpu.VMEM((ni, k), jnp.int32),
            [pltpu.SemaphoreType.DMA] * ni,
            [pltpu.SemaphoreType.DMA] * ni,
            [pltpu.SemaphoreType.DMA] * nr,
            [pltpu.SemaphoreType.DMA] * nr,
        ],
    )
    def scat_kernel(vh_hbm, ei_hbm, zeros_hbm, out_hbm,
                    accum, rows, sidx, didx, sis, dis, gss, sss):
        c = lax.axis_index("c")
        s = lax.axis_index("s")
        base = s * e_per

        def idx_copy(j, q):
            off = base + j * k
            pltpu.async_copy(ei_hbm.at[pl.ds(off, k)], sidx.at[q], sis[q])
            pltpu.async_copy(ei_hbm.at[pl.ds(e + off, k)], didx.at[q], dis[q])

        def wait_sidx(q):
            pltpu.make_async_copy(ei_hbm.at[pl.ds(base, k)], sidx.at[q], sis[q]).wait()

        def wait_didx(q):
            pltpu.make_async_copy(ei_hbm.at[pl.ds(base, k)], didx.at[q], dis[q]).wait()

        def gather(q, b):
            pltpu.async_copy(vh_hbm.at[c].at[sidx.at[q]], rows.at[b], gss[b])

        def wait_gather(q, b):
            pltpu.make_async_copy(vh_hbm.at[c].at[sidx.at[q]], rows.at[b], gss[b]).wait()

        def scat(q, b):
            pltpu.async_copy(rows.at[b], accum.at[didx.at[q]], sss[b], add=True)

        def wait_scat(q, b):
            pltpu.make_async_copy(rows.at[b], accum.at[didx.at[q]], sss[b]).wait()

        for q in range(ni):
            idx_copy(q, q)
        pltpu.sync_copy(zeros_hbm.at[pl.ds(s * stripe, stripe)],
                        accum.at[pl.ds(s * stripe, stripe)])
        plsc.subcore_barrier()
        for b in range(nr):
            wait_sidx(b)
            gather(b, b)

        def outer(g, carry):
            j0 = g * ni
            for t in range(ni):
                q = t
                b = t % nr
                qn = (t + nr) % ni
                wait_gather(q, b)
                wait_didx(q)
                scat(q, b)
                wait_scat(q, b)
                idx_copy(j0 + t + ni, q)
                wait_sidx(qn)
                gather(qn, b)
            return carry

        lax.fori_loop(0, ngrp, outer, 0)
        for t in range(ni):
            q = t
            b = t % nr
            qn = (t + nr) % ni
            wait_gather(q, b)
            wait_didx(q)
            scat(q, b)
            wait_scat(q, b)
            if t < nr:
                wait_sidx(qn)
                gather(qn, b)
        plsc.subcore_barrier()
        pltpu.sync_copy(accum.at[pl.ds(s * stripe, stripe)],
                        out_hbm.at[c].at[pl.ds(s * stripe, stripe)])

    return scat_kernel(vh, ei_flat, zeros_h)


def _dinv_from(d_ref):
    deg = d_ref[0] + d_ref[1] + 1.0
    return lax.rsqrt(jnp.maximum(deg, 1e-12))


def _tc1_body(x_ref, w_ref, d_ref, o_ref):
    dinv = _dinv_from(d_ref)
    u = jnp.dot(x_ref[...], w_ref[...], preferred_element_type=jnp.float32)
    o_ref[...] = (dinv * u)[None]


def _tc1(x, w1, deg2):
    n, f = x.shape
    h = f // 2
    r = 1000
    return pl.pallas_call(
        _tc1_body,
        grid=(NC, n // r),
        in_specs=[
            pl.BlockSpec((r, f), lambda c, i: (i, 0)),
            pl.BlockSpec((f, h), lambda c, i: (0, c)),
            pl.BlockSpec((NC, r, 1), lambda c, i: (0, i, 0)),
        ],
        out_specs=pl.BlockSpec((1, r, h), lambda c, i: (c, i, 0)),
        out_shape=jax.ShapeDtypeStruct((NC, n, h), jnp.float32),
    )(x, w1, deg2)


def _tc2_body(s1_ref, v_ref, d_ref, b_ref, o_ref):
    dinv = _dinv_from(d_ref)
    hact = jnp.maximum(dinv * (s1_ref[0] + v_ref[0]) + b_ref[0], 0.0)
    o_ref[...] = (dinv * hact)[None]


def _tc2(s1, v, deg2, b1r):
    _, n, h = v.shape
    r = 1000
    return pl.pallas_call(
        _tc2_body,
        grid=(NC, n // r),
        in_specs=[
            pl.BlockSpec((1, r, h), lambda c, i: (c, i, 0)),
            pl.BlockSpec((1, r, h), lambda c, i: (c, i, 0)),
            pl.BlockSpec((NC, r, 1), lambda c, i: (0, i, 0)),
            pl.BlockSpec((1, 1, h), lambda c, i: (c, 0, 0)),
        ],
        out_specs=pl.BlockSpec((1, r, h), lambda c, i: (c, i, 0)),
        out_shape=jax.ShapeDtypeStruct((NC, n, h), jnp.float32),
    )(s1, v, deg2, b1r)


def _tc3_body(s2_ref, v2_ref, d_ref, w2_ref, w3_ref, b2_ref, b3_ref,
              mu_ref, ls_ref):
    dinv = _dinv_from(d_ref)
    g0 = dinv * (s2_ref[0] + v2_ref[0])
    g1 = dinv * (s2_ref[1] + v2_ref[1])
    mu_ref[...] = (jnp.dot(g0, w2_ref[0], preferred_element_type=jnp.float32)
                   + jnp.dot(g1, w2_ref[1], preferred_element_type=jnp.float32)
                   + b2_ref[...])
    ls_ref[...] = (jnp.dot(g0, w3_ref[0], preferred_element_type=jnp.float32)
                   + jnp.dot(g1, w3_ref[1], preferred_element_type=jnp.float32)
                   + b3_ref[...])


def _tc3(s2, v2, deg2, w2r, w3r, b2r, b3r):
    _, n, h = v2.shape
    r = 1000
    return pl.pallas_call(
        _tc3_body,
        grid=(n // r,),
        in_specs=[
            pl.BlockSpec((NC, r, h), lambda i: (0, i, 0)),
            pl.BlockSpec((NC, r, h), lambda i: (0, i, 0)),
            pl.BlockSpec((NC, r, 1), lambda i: (0, i, 0)),
            pl.BlockSpec((NC, h, h), lambda i: (0, 0, 0)),
            pl.BlockSpec((NC, h, h), lambda i: (0, 0, 0)),
            pl.BlockSpec((1, h), lambda i: (0, 0)),
            pl.BlockSpec((1, h), lambda i: (0, 0)),
        ],
        out_specs=[
            pl.BlockSpec((r, h), lambda i: (i, 0)),
            pl.BlockSpec((r, h), lambda i: (i, 0)),
        ],
        out_shape=[
            jax.ShapeDtypeStruct((n, h), jnp.float32),
            jax.ShapeDtypeStruct((n, h), jnp.float32),
        ],
    )(s2, v2, deg2, w2r, w3r, b2r, b3r)


def kernel(x, edge_idx, W1, b1, W2, b2, W3, b3):
    n, f = x.shape
    h = f // 2
    ei_flat = edge_idx.astype(jnp.int32).reshape(-1)
    e = ei_flat.shape[0] // 2

    npad = -(-n // (NS * 128)) * (NS * 128)
    zeros_h = jnp.zeros((npad, h), jnp.float32)
    zeros1 = jnp.zeros((npad,), jnp.float32)
    ones1 = jnp.ones((40,), jnp.float32)
    b1r = b1.reshape(NC, 1, h)
    w2r = W2.reshape(NC, h, h)
    w3r = W3.reshape(NC, h, h)
    b2r = b2.reshape(1, h)
    b3r = b3.reshape(1, h)

    deg2 = _sc_degree(ei_flat, zeros1, ones1, npad).reshape(NC, npad, 1)
    v = _tc1(x, W1, deg2)
    s1 = _sc_scatter_add(v, ei_flat, zeros_h, npad)
    v2 = _tc2(s1, v, deg2, b1r)
    s2 = _sc_scatter_add(v2, ei_flat, zeros_h, npad)
    mu, log_std = _tc3(s2, v2, deg2, w2r, w3r, b2r, b3r)
    return (mu, log_std)


# speedup vs baseline: 1.5726x; 1.0004x over previous
import functools

import jax
import jax.numpy as jnp
from jax import lax
from jax.experimental import pallas as pl
from jax.experimental.pallas import tpu as pltpu
from jax.experimental.pallas import tpu_sc as plsc

NC = 2
NS = 16


def _chunk(n, cap=128):
    for k in range(cap - cap % 8, 0, -8):
        if n % k == 0:
            return k
    raise ValueError(f"no multiple-of-8 chunk for {n}")


def _div(n, cap=128):
    for k in range(min(cap, n), 0, -1):
        if n % k == 0:
            return k
    return 1


def _sc_degree(ei_flat, zeros1, ones1, npad):
    e = ei_flat.shape[0] // 2
    e_per = e // (NC * NS)
    k = 40
    nbuf = 5
    nloops = e_per // k
    ngrp = nloops // nbuf
    stripe = npad // NS
    mesh = plsc.VectorSubcoreMesh(core_axis_name="c", subcore_axis_name="s")

    @functools.partial(
        pl.kernel,
        out_type=jax.ShapeDtypeStruct((NC, npad), jnp.float32),
        mesh=mesh,
        scratch_types=[
            pltpu.VMEM_SHARED((npad,), jnp.float32),
            pltpu.VMEM((k,), jnp.float32),
            pltpu.VMEM((nbuf, k), jnp.int32),
            [pltpu.SemaphoreType.DMA] * nbuf,
            [pltpu.SemaphoreType.DMA] * nbuf,
        ],
    )
    def deg_kernel(ei_hbm, zeros_hbm, ones_hbm, out_hbm,
                   accum, ones_v, didx, dis, sss):
        c = lax.axis_index("c")
        s = lax.axis_index("s")
        base = e + (c * NS + s) * e_per

        def idx_copy(j, b):
            pltpu.async_copy(ei_hbm.at[pl.ds(base + j * k, k)], didx.at[b], dis[b])

        def wait_idx(b):
            pltpu.make_async_copy(ei_hbm.at[pl.ds(base, k)], didx.at[b], dis[b]).wait()

        def scat(b):
            pltpu.async_copy(ones_v, accum.at[didx.at[b]], sss[b], add=True)

        def wait_scat(b):
            pltpu.make_async_copy(ones_v, accum.at[didx.at[b]], sss[b]).wait()

        for b in range(nbuf):
            idx_copy(b, b)
        pltpu.sync_copy(zeros_hbm.at[pl.ds(s * stripe, stripe)],
                        accum.at[pl.ds(s * stripe, stripe)])
        pltpu.sync_copy(ones_hbm, ones_v)
        plsc.subcore_barrier()

        def outer(g, carry):
            j0 = g * nbuf
            for b in range(nbuf):
                wait_idx(b)
                scat(b)
                wait_scat(b)
                idx_copy(j0 + nbuf + b, b)
            return carry

        lax.fori_loop(0, ngrp - 1, outer, 0)
        for b in range(nbuf):
            wait_idx(b)
            scat(b)
            wait_scat(b)
        plsc.subcore_barrier()
        pltpu.sync_copy(accum.at[pl.ds(s * stripe, stripe)],
                        out_hbm.at[c].at[pl.ds(s * stripe, stripe)])

    return deg_kernel(ei_flat, zeros1, ones1)


def _sc_scatter_add(vh, ei_flat, zeros_h, npad):
    e = ei_flat.shape[0] // 2
    h = vh.shape[2]
    e_per = e // NS
    k = 40
    nr = 5
    ni = 10
    nloops = e_per // k
    ngrp = nloops // ni - 1
    stripe = npad // NS
    mesh = plsc.VectorSubcoreMesh(core_axis_name="c", subcore_axis_name="s")

    @functools.partial(
        pl.kernel,
        out_type=jax.ShapeDtypeStruct((NC, npad, h), jnp.float32),
        mesh=mesh,
        scratch_types=[
            pltpu.VMEM_SHARED((npad, h), jnp.float32),
            pltpu.VMEM((nr, k, h), jnp.float32),
            pltpu.VMEM((ni, k), jnp.int32),
            pltpu.VMEM((ni, k), jnp.int32),
            [pltpu.SemaphoreType.DMA] * ni,
            [pltpu.SemaphoreType.DMA] * ni,
            [pltpu.SemaphoreType.DMA] * nr,
            [pltpu.SemaphoreType.DMA] * nr,
        ],
    )
    def scat_kernel(vh_hbm, ei_hbm, zeros_hbm, out_hbm,
                    accum, rows, sidx, didx, sis, dis, gss, sss):
        c = lax.axis_index("c")
        s = lax.axis_index("s")
        base = s * e_per

        def idx_copy(j, q):
            off = base + j * k
            pltpu.async_copy(ei_hbm.at[pl.ds(off, k)], sidx.at[q], sis[q])
            pltpu.async_copy(ei_hbm.at[pl.ds(e + off, k)], didx.at[q], dis[q])

        def wait_sidx(q):
            pltpu.make_async_copy(ei_hbm.at[pl.ds(base, k)], sidx.at[q], sis[q]).wait()

        def wait_didx(q):
            pltpu.make_async_copy(ei_hbm.at[pl.ds(base, k)], didx.at[q], dis[q]).wait()

        def gather(q, b):
            pltpu.async_copy(vh_hbm.at[c].at[sidx.at[q]], rows.at[b], gss[b])

        def wait_gather(q, b):
            pltpu.make_async_copy(vh_hbm.at[c].at[sidx.at[q]], rows.at[b], gss[b]).wait()

        def scat(q, b):
            pltpu.async_copy(rows.at[b], accum.at[didx.at[q]], sss[b], add=True)

        def wait_scat(q, b):
            pltpu.make_async_copy(rows.at[b], accum.at[didx.at[q]], sss[b]).wait()

        for q in range(ni):
            idx_copy(q, q)
        pltpu.sync_copy(zeros_hbm.at[pl.ds(s * stripe, stripe)],
                        accum.at[pl.ds(s * stripe, stripe)])
        plsc.subcore_barrier()
        for b in range(nr):
            wait_sidx(b)
            gather(b, b)

        def outer(g, carry):
            j0 = g * ni
            for t in range(ni):
                q = t
                b = t % nr
                qn = (t + nr) % ni
                wait_gather(q, b)
                wait_didx(q)
                scat(q, b)
                wait_scat(q, b)
                idx_copy(j0 + t + ni, q)
                wait_sidx(qn)
                gather(qn, b)
            return carry

        lax.fori_loop(0, ngrp, outer, 0)
        for t in range(ni):
            q = t
            b = t % nr
            qn = (t + nr) % ni
            wait_gather(q, b)
            wait_didx(q)
            scat(q, b)
            wait_scat(q, b)
            if t < nr:
                wait_sidx(qn)
                gather(qn, b)
        plsc.subcore_barrier()
        pltpu.sync_copy(accum.at[pl.ds(s * stripe, stripe)],
                        out_hbm.at[c].at[pl.ds(s * stripe, stripe)])

    return scat_kernel(vh, ei_flat, zeros_h)


def _dinv_from(d_ref):
    deg = d_ref[0] + d_ref[1] + 1.0
    return lax.rsqrt(jnp.maximum(deg, 1e-12))


def _tc1_body(x_ref, w_ref, d_ref, o_ref):
    dinv = _dinv_from(d_ref)
    u = jnp.dot(x_ref[...], w_ref[...], preferred_element_type=jnp.float32)
    o_ref[...] = (dinv * u)[None]


def _tc1(x, w1, deg2):
    n, f = x.shape
    h = f // 2
    r = 1000
    return pl.pallas_call(
        _tc1_body,
        grid=(NC, n // r),
        in_specs=[
            pl.BlockSpec((r, f), lambda c, i: (i, 0)),
            pl.BlockSpec((f, h), lambda c, i: (0, c)),
            pl.BlockSpec((NC, r, 1), lambda c, i: (0, i, 0)),
        ],
        out_specs=pl.BlockSpec((1, r, h), lambda c, i: (c, i, 0)),
        out_shape=jax.ShapeDtypeStruct((NC, n, h), jnp.float32),
    )(x, w1, deg2)


def _tc2_body(s1_ref, v_ref, d_ref, b_ref, o_ref):
    dinv = _dinv_from(d_ref)
    hact = jnp.maximum(dinv * (s1_ref[0] + v_ref[0]) + b_ref[0], 0.0)
    o_ref[...] = (dinv * hact)[None]


def _tc2(s1, v, deg2, b1r):
    _, n, h = v.shape
    r = 1000
    return pl.pallas_call(
        _tc2_body,
        grid=(NC, n // r),
        in_specs=[
            pl.BlockSpec((1, r, h), lambda c, i: (c, i, 0)),
            pl.BlockSpec((1, r, h), lambda c, i: (c, i, 0)),
            pl.BlockSpec((NC, r, 1), lambda c, i: (0, i, 0)),
            pl.BlockSpec((1, 1, h), lambda c, i: (c, 0, 0)),
        ],
        out_specs=pl.BlockSpec((1, r, h), lambda c, i: (c, i, 0)),
        out_shape=jax.ShapeDtypeStruct((NC, n, h), jnp.float32),
    )(s1, v, deg2, b1r)


def _tc3_body(s2_ref, v2_ref, d_ref, w2_ref, w3_ref, b2_ref, b3_ref,
              mu_ref, ls_ref):
    dinv = _dinv_from(d_ref)
    g0 = dinv * (s2_ref[0] + v2_ref[0])
    g1 = dinv * (s2_ref[1] + v2_ref[1])
    mu_ref[...] = (jnp.dot(g0, w2_ref[0], preferred_element_type=jnp.float32)
                   + jnp.dot(g1, w2_ref[1], preferred_element_type=jnp.float32)
                   + b2_ref[...])
    ls_ref[...] = (jnp.dot(g0, w3_ref[0], preferred_element_type=jnp.float32)
                   + jnp.dot(g1, w3_ref[1], preferred_element_type=jnp.float32)
                   + b3_ref[...])


def _tc3(s2, v2, deg2, w2r, w3r, b2r, b3r):
    _, n, h = v2.shape
    r = 1000
    return pl.pallas_call(
        _tc3_body,
        grid=(n // r,),
        in_specs=[
            pl.BlockSpec((NC, r, h), lambda i: (0, i, 0)),
            pl.BlockSpec((NC, r, h), lambda i: (0, i, 0)),
            pl.BlockSpec((NC, r, 1), lambda i: (0, i, 0)),
            pl.BlockSpec((NC, h, h), lambda i: (0, 0, 0)),
            pl.BlockSpec((NC, h, h), lambda i: (0, 0, 0)),
            pl.BlockSpec((1, h), lambda i: (0, 0)),
            pl.BlockSpec((1, h), lambda i: (0, 0)),
        ],
        out_specs=[
            pl.BlockSpec((r, h), lambda i: (i, 0)),
            pl.BlockSpec((r, h), lambda i: (i, 0)),
        ],
        out_shape=[
            jax.ShapeDtypeStruct((n, h), jnp.float32),
            jax.ShapeDtypeStruct((n, h), jnp.float32),
        ],
    )(s2, v2, deg2, w2r, w3r, b2r, b3r)


def kernel(x, edge_idx, W1, b1, W2, b2, W3, b3):
    n, f = x.shape
    h = f // 2
    ei_flat = edge_idx.astype(jnp.int32).reshape(-1)
    e = ei_flat.shape[0] // 2

    npad = -(-n // (NS * 128)) * (NS * 128)
    zeros_h = jnp.zeros((npad, h), jnp.float32)
    zeros1 = jnp.zeros((npad,), jnp.float32)
    ones1 = jnp.ones((40,), jnp.float32)
    b1r = b1.reshape(NC, 1, h)
    w2r = W2.reshape(NC, h, h)
    w3r = W3.reshape(NC, h, h)
    b2r = b2.reshape(1, h)
    b3r = b3.reshape(1, h)

    deg2 = _sc_degree(ei_flat, zeros1, ones1, npad).reshape(NC, npad, 1)
    v = _tc1(x, W1, deg2)
    s1 = _sc_scatter_add(v, ei_flat, zeros_h, npad)
    v2 = _tc2(s1, v, deg2, b1r)
    s2 = _sc_scatter_add(v2, ei_flat, zeros_h, npad)
    mu, log_std = _tc3(s2, v2, deg2, w2r, w3r, b2r, b3r)
    return (mu, log_std)
